# Initial kernel scaffold; baseline (speedup 1.0000x reference)
import jax
import jax.numpy as jnp
from jax.experimental import pallas as pl


def kernel(x, weights):
    raise NotImplementedError("write your pallas kernel here")


# speedup vs baseline: 34.1878x; 34.1878x over previous
import jax
import jax.numpy as jnp
from jax.experimental import pallas as pl

GRID_ROWS = 64
GRID_COLS = 128
N_CELLS = GRID_ROWS * GRID_COLS
DIM = 256

BM = 1024
BN = 2048


def _dist_kernel(x_ref, w_ref, out_ref):
    x = x_ref[...]
    w = w_ref[...]
    g = jax.lax.dot_general(
        x, w,
        dimension_numbers=(((1,), (1,)), ((), ())),
        preferred_element_type=jnp.float32,
    )
    x2 = jnp.sum(x * x, axis=1, keepdims=True)
    w2 = jnp.sum(w * w, axis=1, keepdims=True).T
    out_ref[...] = x2 + w2 - 2.0 * g


def kernel(x, weights):
    if x.ndim == 1:
        x = x[None, :]
    b = x.shape[0]
    w2d = weights.reshape(N_CELLS, DIM)

    bm = min(BM, b)
    grid = (pl.cdiv(b, bm), N_CELLS // BN)

    out = pl.pallas_call(
        _dist_kernel,
        grid=grid,
        in_specs=[
            pl.BlockSpec((bm, DIM), lambda i, j: (i, 0)),
            pl.BlockSpec((BN, DIM), lambda i, j: (j, 0)),
        ],
        out_specs=pl.BlockSpec((bm, BN), lambda i, j: (i, j)),
        out_shape=jax.ShapeDtypeStruct((b, N_CELLS), jnp.float32),
    )(x, w2d)
    return out.reshape(b, GRID_ROWS, GRID_COLS)
